# baseline (speedup 1.0000x reference)
import functools

import jax
import jax.numpy as jnp
from jax import lax
from jax.experimental import pallas as pl
from jax.experimental.pallas import tpu as pltpu
from jax.experimental.pallas import tpu_sc as plsc

_D = 32
_B1 = 4096
_B2 = 200
_B = _B1 * _B2
_SCALE = float(_D) ** 0.5

_info = plsc.get_sparse_core_info()
_NC, _NS, _L = _info.num_cores, _info.num_subcores, _info.num_lanes
_NW = _NC * _NS

_GRP = 128
_TOK = 512
_GPB = _TOK // _GRP
_BLK_PER_ROW = _B1 // _TOK
_NBLK = _B // _TOK
_BPW = _NBLK // _NW
_NP = _BPW // 2

_mesh = plsc.VectorSubcoreMesh(core_axis_name="c", subcore_axis_name="s")


@functools.partial(
    pl.kernel,
    mesh=_mesh,
    out_type=jax.ShapeDtypeStruct((_B2, _D // 8, _B1 // _GRP, 8, _GRP),
                                  jnp.float32),
    scratch_types=[
        pltpu.VMEM((_GPB, _GRP), jnp.int32),
        pltpu.VMEM((_GPB, _GRP), jnp.int32),
        pltpu.VMEM((_TOK, _D), jnp.float32),
        pltpu.VMEM((_TOK, _D), jnp.float32),
        pltpu.VMEM((_D, _TOK), jnp.float32),
        pltpu.VMEM((_D, _TOK), jnp.float32),
        pltpu.SemaphoreType.DMA,
        pltpu.SemaphoreType.DMA,
        pltpu.SemaphoreType.DMA,
        pltpu.SemaphoreType.DMA,
        pltpu.SemaphoreType.DMA,
    ],
    compiler_params=pltpu.CompilerParams(use_tc_tiling_on_sc=False,
                                         needs_layout_passes=False),
)
def _emb_lookup(table_hbm, x3_hbm, out_hbm, idx_a, idx_b, rows_a, rows_b,
                tbuf_a, tbuf_b, gsem_a, gsem_b, isem, osem_a, osem_b):
    wid = lax.axis_index("s") * _NC + lax.axis_index("c")
    first = wid * _BPW

    def loc(t):
        f = first + t
        return f // _BLK_PER_ROW, lax.rem(f, _BLK_PER_ROW)

    def idx_copy(t, idx_v):
        b2, bb = loc(t)
        return pltpu.make_async_copy(
            x3_hbm.at[b2, pl.ds(bb * _GPB, _GPB)], idx_v, isem)

    def gather_descs(idx_v, rows_v, gsem):
        return [
            pltpu.make_async_copy(
                table_hbm.at[idx_v.at[j]],
                rows_v.at[pl.ds(j * _GRP, _GRP)],
                gsem,
            )
            for j in range(_GPB)
        ]

    def out_descs(t, tbuf, osem):
        b2, bb = loc(t)
        return [
            pltpu.make_async_copy(
                tbuf.at[pl.ds(dt * 8, 8), pl.ds(btl * _GRP, _GRP)],
                out_hbm.at[b2, dt, bb * _GPB + btl],
                osem,
            )
            for dt in range(_D // 8) for btl in range(_GPB)
        ]

    lane = jnp.arange(_L, dtype=jnp.int32)
    dskew = [[(h * _L + ((lane + s) & (_L - 1))).astype(jnp.int32)
              for s in range(_L)] for h in range(2)]

    def compute(rows_v, tbuf):
        def body(tt, c):
            tokvec = lane + tt * _L
            vals = [
                plsc.load_gather(rows_v, [tokvec, dskew[h][s]]) * _SCALE
                for h in range(2) for s in range(_L)
            ]
            for (h, s), v in zip(
                    [(h, s) for h in range(2) for s in range(_L)], vals):
                plsc.store_scatter(tbuf, [dskew[h][s], tokvec], v)
            return c

        lax.fori_loop(0, _TOK // _L, body, 0)

    idx_copy(0, idx_a).start()
    idx_copy(0, idx_a).wait()
    for desc in gather_descs(idx_a, rows_a, gsem_a):
        desc.start()
    idx_copy(1, idx_b).start()
    idx_copy(1, idx_b).wait()
    for desc in gather_descs(idx_b, rows_b, gsem_b):
        desc.start()

    def half(p, t, idx_v, rows_v, tbuf, gsem, osem):
        for desc in gather_descs(idx_v, rows_v, gsem):
            desc.wait()

        @pl.when(p + 1 < _NP)
        def _():
            idx_copy(t + 2, idx_v).start()

        @pl.when(p >= 1)
        def _():
            for desc in out_descs(t - 2, tbuf, osem):
                desc.wait()

        compute(rows_v, tbuf)
        for desc in out_descs(t, tbuf, osem):
            desc.start()

        @pl.when(p + 1 < _NP)
        def _():
            idx_copy(t + 2, idx_v).wait()
            for desc in gather_descs(idx_v, rows_v, gsem):
                desc.start()

    def pair_body(p, carry):
        a = p * 2
        half(p, a, idx_a, rows_a, tbuf_a, gsem_a, osem_a)
        half(p, a + 1, idx_b, rows_b, tbuf_b, gsem_b, osem_b)
        return carry

    lax.fori_loop(0, _NP, pair_body, 0)

    for desc in out_descs(_BPW - 2, tbuf_a, osem_a):
        desc.wait()
    for desc in out_descs(_BPW - 1, tbuf_b, osem_b):
        desc.wait()


def kernel(x, table):
    x3 = jnp.transpose(x).reshape(_B2, _B1 // _GRP, _GRP).astype(jnp.int32)
    out5 = _emb_lookup(table, x3)
    out_t = jnp.transpose(out5, (0, 1, 3, 2, 4)).reshape(_B2, _D, _B1)
    return jnp.transpose(out_t, (2, 0, 1))


# speedup vs baseline: 1.2249x; 1.2249x over previous
import functools

import jax
import jax.numpy as jnp
from jax import lax
from jax.experimental import pallas as pl
from jax.experimental.pallas import tpu as pltpu
from jax.experimental.pallas import tpu_sc as plsc

_D = 32
_B1 = 4096
_B2 = 200
_B = _B1 * _B2
_SCALE = float(_D) ** 0.5

_info = plsc.get_sparse_core_info()
_NC, _NS, _L = _info.num_cores, _info.num_subcores, _info.num_lanes
_NW = _NC * _NS

_GRP = 128
_TOK = 512
_GPB = _TOK // _GRP
_BLK_PER_ROW = _B1 // _TOK
_NBLK = _B // _TOK
_BPW = _NBLK // _NW
_NP = _BPW // 2

_mesh = plsc.VectorSubcoreMesh(core_axis_name="c", subcore_axis_name="s")


@functools.partial(
    pl.kernel,
    mesh=_mesh,
    out_type=jax.ShapeDtypeStruct((_B2, _D // 8, _B1 // _GRP, 8, _GRP),
                                  jnp.float32),
    scratch_types=[
        pltpu.VMEM((_GPB, _GRP), jnp.int32),
        pltpu.VMEM((_GPB, _GRP), jnp.int32),
        pltpu.VMEM((_TOK, _D), jnp.float32),
        pltpu.VMEM((_TOK, _D), jnp.float32),
        pltpu.VMEM((_D, _TOK), jnp.float32),
        pltpu.VMEM((_D, _TOK), jnp.float32),
        pltpu.SemaphoreType.DMA,
        pltpu.SemaphoreType.DMA,
        pltpu.SemaphoreType.DMA,
        pltpu.SemaphoreType.DMA,
        pltpu.SemaphoreType.DMA,
    ],
    compiler_params=pltpu.CompilerParams(use_tc_tiling_on_sc=False,
                                         needs_layout_passes=False),
)
def _emb_lookup(table_hbm, x3_hbm, out_hbm, idx_a, idx_b, rows_a, rows_b,
                tbuf_a, tbuf_b, gsem_a, gsem_b, isem, osem_a, osem_b):
    wid = lax.axis_index("s") * _NC + lax.axis_index("c")
    first = wid * _BPW

    def loc(t):
        f = first + t
        return f // _BLK_PER_ROW, lax.rem(f, _BLK_PER_ROW)

    def idx_copy(t, idx_v):
        b2, bb = loc(t)
        return pltpu.make_async_copy(
            x3_hbm.at[b2, pl.ds(bb * _GPB, _GPB)], idx_v, isem)

    def gather_descs(idx_v, rows_v, gsem):
        return [
            pltpu.make_async_copy(
                table_hbm.at[idx_v.at[j]],
                rows_v.at[pl.ds(j * _GRP, _GRP)],
                gsem,
            )
            for j in range(_GPB)
        ]

    def out_descs(t, tbuf, osem):
        b2, bb = loc(t)
        return [
            pltpu.make_async_copy(
                tbuf.at[pl.ds(dt * 8, 8), pl.ds(btl * _GRP, _GRP)],
                out_hbm.at[b2, dt, bb * _GPB + btl],
                osem,
            )
            for dt in range(_D // 8) for btl in range(_GPB)
        ]

    def xform_idx(idx_v):
        for j in range(_GPB):
            for g in range(_GRP // _L):
                sl = pl.ds(g * _L, _L)
                m = idx_v[j, sl]
                idx_v[j, sl] = (
                    (m & jnp.int32(-4096))
                    | lax.shift_left(m & jnp.int32(1023), jnp.int32(2))
                    | (lax.shift_right_logical(m, jnp.int32(10))
                       & jnp.int32(3)))

    lane = jnp.arange(_L, dtype=jnp.int32)
    dskew = [[(h * _L + ((lane + s) & (_L - 1))).astype(jnp.int32)
              for s in range(_L)] for h in range(2)]

    def compute(rows_v, tbuf):
        def body(tt, c):
            tokvec = lane + tt * _L
            vals = [
                plsc.load_gather(rows_v, [tokvec, dskew[h][s]]) * _SCALE
                for h in range(2) for s in range(_L)
            ]
            for (h, s), v in zip(
                    [(h, s) for h in range(2) for s in range(_L)], vals):
                plsc.store_scatter(tbuf, [dskew[h][s], tokvec], v)
            return c

        lax.fori_loop(0, _TOK // _L, body, 0)

    idx_copy(0, idx_a).start()
    idx_copy(0, idx_a).wait()
    xform_idx(idx_a)
    for desc in gather_descs(idx_a, rows_a, gsem_a):
        desc.start()
    idx_copy(1, idx_b).start()
    idx_copy(1, idx_b).wait()
    xform_idx(idx_b)
    for desc in gather_descs(idx_b, rows_b, gsem_b):
        desc.start()

    def half(p, t, idx_v, rows_v, tbuf, gsem, osem):
        for desc in gather_descs(idx_v, rows_v, gsem):
            desc.wait()

        @pl.when(p + 1 < _NP)
        def _():
            idx_copy(t + 2, idx_v).start()

        @pl.when(p >= 1)
        def _():
            for desc in out_descs(t - 2, tbuf, osem):
                desc.wait()

        compute(rows_v, tbuf)
        for desc in out_descs(t, tbuf, osem):
            desc.start()

        @pl.when(p + 1 < _NP)
        def _():
            idx_copy(t + 2, idx_v).wait()
            xform_idx(idx_v)
            for desc in gather_descs(idx_v, rows_v, gsem):
                desc.start()

    def pair_body(p, carry):
        a = p * 2
        half(p, a, idx_a, rows_a, tbuf_a, gsem_a, osem_a)
        half(p, a + 1, idx_b, rows_b, tbuf_b, gsem_b, osem_b)
        return carry

    lax.fori_loop(0, _NP, pair_body, 0)

    for desc in out_descs(_BPW - 2, tbuf_a, osem_a):
        desc.wait()
    for desc in out_descs(_BPW - 1, tbuf_b, osem_b):
        desc.wait()


def _tc_detile_body(t_ref, o_ref):
    t = jnp.transpose(t_ref[...])
    for q in range(4):
        o_ref[:, 32 * q:32 * (q + 1)] = t[1024 * q:1024 * (q + 1), :]


_tc_detile = pl.pallas_call(
    _tc_detile_body,
    grid=(245,),
    in_specs=[pl.BlockSpec((_D, 4096), lambda i: (0, i))],
    out_specs=pl.BlockSpec((1024, 128), lambda i: (i, 0)),
    out_shape=jax.ShapeDtypeStruct((250880, 128), jnp.float32),
)


def kernel(x, table):
    x3 = jnp.transpose(x).reshape(_B2, _B1 // _GRP, _GRP).astype(jnp.int32)
    table = _tc_detile(jnp.transpose(table)).reshape(1003520, _D)
    out5 = _emb_lookup(table, x3)
    out_t = jnp.transpose(out5, (0, 1, 3, 2, 4)).reshape(_B2, _D, _B1)
    return jnp.transpose(out_t, (2, 0, 1))


# speedup vs baseline: 1.4472x; 1.1814x over previous
import functools

import jax
import jax.numpy as jnp
from jax import lax
from jax.experimental import pallas as pl
from jax.experimental.pallas import tpu as pltpu
from jax.experimental.pallas import tpu_sc as plsc

_D = 32
_B1 = 4096
_B2 = 200
_B = _B1 * _B2
_SCALE = float(_D) ** 0.5

_info = plsc.get_sparse_core_info()
_NC, _NS, _L = _info.num_cores, _info.num_subcores, _info.num_lanes
_NW = _NC * _NS

_GRP = 128
_TOK = 512
_GPB = _TOK // _GRP
_BLK_PER_ROW = _B1 // _TOK
_NBLK = _B // _TOK
_BPW = _NBLK // _NW
_NP = _BPW // 2

_mesh = plsc.VectorSubcoreMesh(core_axis_name="c", subcore_axis_name="s")


@functools.partial(
    pl.kernel,
    mesh=_mesh,
    out_type=jax.ShapeDtypeStruct((_B2, _D // 8, _B1 // _GRP, 8, _GRP),
                                  jnp.float32),
    scratch_types=[
        pltpu.VMEM((_GPB, _GRP), jnp.int32),
        pltpu.VMEM((_GPB, _GRP), jnp.int32),
        pltpu.VMEM((_TOK, _D), jnp.float32),
        pltpu.VMEM((_TOK, _D), jnp.float32),
        pltpu.VMEM((_D, _TOK), jnp.float32),
        pltpu.VMEM((_D, _TOK), jnp.float32),
        pltpu.SemaphoreType.DMA,
        pltpu.SemaphoreType.DMA,
        pltpu.SemaphoreType.DMA,
        pltpu.SemaphoreType.DMA,
        pltpu.SemaphoreType.DMA,
    ],
    compiler_params=pltpu.CompilerParams(use_tc_tiling_on_sc=False,
                                         needs_layout_passes=False),
)
def _emb_lookup(table_hbm, x3_hbm, out_hbm, idx_a, idx_b, rows_a, rows_b,
                tbuf_a, tbuf_b, gsem_a, gsem_b, isem, osem_a, osem_b):
    wid = lax.axis_index("s") * _NC + lax.axis_index("c")
    first = wid * _BPW

    def loc(t):
        f = first + t
        return f // _BLK_PER_ROW, lax.rem(f, _BLK_PER_ROW)

    def idx_copy(t, idx_v):
        b2, bb = loc(t)
        return pltpu.make_async_copy(
            x3_hbm.at[b2, pl.ds(bb * _GPB, _GPB)], idx_v, isem)

    def gather_descs(idx_v, rows_v, gsem):
        return [
            pltpu.make_async_copy(
                table_hbm.at[idx_v.at[j]],
                rows_v.at[pl.ds(j * _GRP, _GRP)],
                gsem,
            )
            for j in range(_GPB)
        ]

    def out_descs(t, tbuf, osem):
        b2, bb = loc(t)
        return [
            pltpu.make_async_copy(
                tbuf.at[pl.ds(dt * 8, 8), pl.ds(btl * _GRP, _GRP)],
                out_hbm.at[b2, dt, bb * _GPB + btl],
                osem,
            )
            for dt in range(_D // 8) for btl in range(_GPB)
        ]

    lane = jnp.arange(_L, dtype=jnp.int32)
    dskew = [[(h * _L + ((lane + s) & (_L - 1))).astype(jnp.int32)
              for s in range(_L)] for h in range(2)]

    def compute(rows_v, tbuf):
        def body(tt, c):
            tokvec = lane + tt * _L
            vals = [
                plsc.load_gather(rows_v, [tokvec, dskew[h][s]]) * _SCALE
                for h in range(2) for s in range(_L)
            ]
            for (h, s), v in zip(
                    [(h, s) for h in range(2) for s in range(_L)], vals):
                plsc.store_scatter(tbuf, [dskew[h][s], tokvec], v)
            return c

        lax.fori_loop(0, _TOK // _L, body, 0)

    idx_copy(0, idx_a).start()
    idx_copy(0, idx_a).wait()
    for desc in gather_descs(idx_a, rows_a, gsem_a):
        desc.start()
    idx_copy(1, idx_b).start()
    idx_copy(1, idx_b).wait()
    for desc in gather_descs(idx_b, rows_b, gsem_b):
        desc.start()

    def half(p, t, idx_v, rows_v, tbuf, gsem, osem):
        for desc in gather_descs(idx_v, rows_v, gsem):
            desc.wait()

        @pl.when(p + 1 < _NP)
        def _():
            idx_copy(t + 2, idx_v).start()

        @pl.when(p >= 1)
        def _():
            for desc in out_descs(t - 2, tbuf, osem):
                desc.wait()

        compute(rows_v, tbuf)
        for desc in out_descs(t, tbuf, osem):
            desc.start()

        @pl.when(p + 1 < _NP)
        def _():
            idx_copy(t + 2, idx_v).wait()
            for desc in gather_descs(idx_v, rows_v, gsem):
                desc.start()

    def pair_body(p, carry):
        a = p * 2
        half(p, a, idx_a, rows_a, tbuf_a, gsem_a, osem_a)
        half(p, a + 1, idx_b, rows_b, tbuf_b, gsem_b, osem_b)
        return carry

    lax.fori_loop(0, _NP, pair_body, 0)

    for desc in out_descs(_BPW - 2, tbuf_a, osem_a):
        desc.wait()
    for desc in out_descs(_BPW - 1, tbuf_b, osem_b):
        desc.wait()


def _tc_detile_body(t_ref, o_ref):
    t = jnp.transpose(t_ref[...])
    for q in range(4):
        o_ref[:, 32 * q:32 * (q + 1)] = t[1024 * q:1024 * (q + 1), :]


_tc_detile = pl.pallas_call(
    _tc_detile_body,
    grid=(245,),
    in_specs=[pl.BlockSpec((_D, 4096), lambda i: (0, i))],
    out_specs=pl.BlockSpec((1024, 128), lambda i: (i, 0)),
    out_shape=jax.ShapeDtypeStruct((250880, 128), jnp.float32),
)


def kernel(x, table):
    m = jnp.transpose(x).reshape(_B2, _B1 // _GRP, _GRP).astype(jnp.int32)
    x3 = ((m & jnp.int32(-4096))
          | jnp.left_shift(m & jnp.int32(1023), jnp.int32(2))
          | (jnp.right_shift(m, jnp.int32(10)) & jnp.int32(3)))
    table = _tc_detile(jnp.transpose(table)).reshape(1003520, _D)
    out5 = _emb_lookup(table, x3)
    out_t = jnp.transpose(out5, (0, 1, 3, 2, 4)).reshape(_B2, _D, _B1)
    return jnp.transpose(out_t, (2, 0, 1))
